# baseline (speedup 1.0000x reference)
import functools

import jax
import jax.numpy as jnp
from jax import lax
from jax.experimental import pallas as pl
from jax.experimental.pallas import tpu as pltpu
from jax.experimental.pallas import tpu_sc as plsc

_NBUF = 4


@functools.lru_cache(maxsize=None)
def _build(N, V, D, NC, NS):
    NW = NC * NS
    per_w = N // NW
    C = 160
    iters = per_w // C
    outer = iters // _NBUF
    assert iters % _NBUF == 0 and outer >= 2

    mesh = plsc.VectorSubcoreMesh(core_axis_name="c", subcore_axis_name="s")

    scratch = (
        [pltpu.VMEM((C,), jnp.int32) for _ in range(_NBUF)]
        + [pltpu.VMEM((C, D), jnp.float32) for _ in range(_NBUF)]
        + [pltpu.SemaphoreType.DMA for _ in range(2 * _NBUF)]
    )

    @functools.partial(
        pl.kernel,
        mesh=mesh,
        out_type=jax.ShapeDtypeStruct((N, D), jnp.float32),
        scratch_types=scratch,
    )
    def gather_kernel(ids_hbm, table_hbm, out_hbm, *bufs):
        idx = bufs[0:_NBUF]
        rows = bufs[_NBUF:2 * _NBUF]
        sg = bufs[2 * _NBUF:3 * _NBUF]
        ss = bufs[3 * _NBUF:4 * _NBUF]

        wid = lax.axis_index("s") * NC + lax.axis_index("c")
        base = wid * per_w

        def start_gather(b, off):
            pltpu.sync_copy(ids_hbm.at[pl.ds(off, C)], idx[b])
            pltpu.async_copy(table_hbm.at[idx[b]], rows[b], sg[b])

        def wait_gather(b):
            pltpu.make_async_copy(table_hbm.at[idx[b]], rows[b], sg[b]).wait()

        def start_store(b, off):
            pltpu.async_copy(rows[b], out_hbm.at[pl.ds(off, C)], ss[b])

        def wait_store(b, off):
            pltpu.make_async_copy(
                rows[b], out_hbm.at[pl.ds(off, C)], ss[b]).wait()

        for b in range(_NBUF):
            start_gather(b, base + b * C)

        def body(go, carry):
            for b in range(_NBUF):
                off = base + (go * _NBUF + b) * C
                wait_gather(b)
                start_store(b, off)

                @pl.when(go < outer - 1)
                def _():
                    wait_store(b, off)
                    start_gather(b, off + _NBUF * C)

            return carry

        lax.fori_loop(0, outer, body, 0)

        last = base + (iters - _NBUF) * C
        for b in range(_NBUF):
            wait_store(b, last + b * C)

    return gather_kernel


def kernel(input_ids, embedding_matrix):
    B, S = input_ids.shape
    V, D = embedding_matrix.shape
    N = B * S
    info = plsc.get_sparse_core_info()
    fn = _build(N, V, D, info.num_cores, info.num_subcores)
    out = fn(input_ids.reshape(N), embedding_matrix)
    return out.reshape(B, S, D)


# speedup vs baseline: 1.0092x; 1.0092x over previous
import functools

import jax
import jax.numpy as jnp
from jax import lax
from jax.experimental import pallas as pl
from jax.experimental.pallas import tpu as pltpu
from jax.experimental.pallas import tpu_sc as plsc

_NBUF = 4


@functools.lru_cache(maxsize=None)
def _build(N, V, D, NC, NS):
    NW = NC * NS
    per_w = N // NW
    C = 200
    iters = per_w // C
    assert iters % _NBUF == 0 and iters >= 2 * _NBUF

    mesh = plsc.VectorSubcoreMesh(core_axis_name="c", subcore_axis_name="s")

    scratch = (
        [pltpu.VMEM((C,), jnp.int32) for _ in range(_NBUF)]
        + [pltpu.VMEM((C, D), jnp.float32) for _ in range(_NBUF)]
        + [pltpu.SemaphoreType.DMA for _ in range(2 * _NBUF)]
    )

    @functools.partial(
        pl.kernel,
        mesh=mesh,
        out_type=jax.ShapeDtypeStruct((N, D), jnp.float32),
        scratch_types=scratch,
    )
    def gather_kernel(ids_hbm, table_hbm, out_hbm, *bufs):
        idx = bufs[0:_NBUF]
        rows = bufs[_NBUF:2 * _NBUF]
        sg = bufs[2 * _NBUF:3 * _NBUF]
        ss = bufs[3 * _NBUF:4 * _NBUF]

        wid = lax.axis_index("s") * NC + lax.axis_index("c")
        base = wid * per_w

        def start_gather(b, off):
            pltpu.sync_copy(ids_hbm.at[pl.ds(off, C)], idx[b])
            pltpu.async_copy(table_hbm.at[idx[b]], rows[b], sg[b])

        def wait_gather(b):
            pltpu.make_async_copy(table_hbm.at[idx[b]], rows[b], sg[b]).wait()

        def start_store(b, off):
            pltpu.async_copy(rows[b], out_hbm.at[pl.ds(off, C)], ss[b])

        def wait_store(b, off):
            pltpu.make_async_copy(
                rows[b], out_hbm.at[pl.ds(off, C)], ss[b]).wait()

        for b in range(_NBUF):
            start_gather(b, base + b * C)
        for b in range(2):
            wait_gather(b)
            start_store(b, base + b * C)

        def body(g, carry):
            first = base + (g + 1) * (_NBUF * C)
            for b in range(_NBUF):
                off = first + b * C
                wait_store(b, off - _NBUF * C)
                start_gather(b, off)
                jb = (b + 2) % _NBUF
                joff = off - 2 * C
                wait_gather(jb)
                start_store(jb, joff)
            return carry

        lax.fori_loop(0, iters // _NBUF - 1, body, 0)

        for i in (iters - 2, iters - 1):
            b = i % _NBUF
            wait_gather(b)
            start_store(b, base + i * C)
        for i in range(iters - _NBUF, iters):
            b = i % _NBUF
            wait_store(b, base + i * C)

    return gather_kernel


def kernel(input_ids, embedding_matrix):
    B, S = input_ids.shape
    V, D = embedding_matrix.shape
    N = B * S
    info = plsc.get_sparse_core_info()
    fn = _build(N, V, D, info.num_cores, info.num_subcores)
    out = fn(input_ids.reshape(N), embedding_matrix)
    return out.reshape(B, S, D)
